# Initial kernel scaffold; baseline (speedup 1.0000x reference)
import jax
import jax.numpy as jnp
from jax.experimental import pallas as pl


def kernel(x, edge_index, batch, W1, b1, W2, b2):
    raise NotImplementedError("write your pallas kernel here")


# speedup vs baseline: 12.9309x; 12.9309x over previous
import functools

import jax
import jax.numpy as jnp
from jax import lax
from jax.experimental import pallas as pl
from jax.experimental.pallas import tpu as pltpu
from jax.experimental.pallas import tpu_sc as plsc

N = 10000
NPAD = 10240
D = 128
E = 320000
G = 16
NC, NS = 2, 16
NW = NC * NS
EW = E // NW
K = 80
ET = E // NS
NCH2 = ET // K
NH = NPAD // NC
ACC = 5184
AZT = ACC // NS
RTW = NH // NS
BR = 1024
GRID = NPAD // BR


def _deg_body(dst_hbm, out_hbm, dstv, buf, shared):
    c = lax.axis_index("c")
    s = lax.axis_index("s")
    base = c * NH

    @pl.loop(0, AZT)
    def _zero(i):
        buf[i, :] = jnp.zeros((16,), jnp.float32)

    pltpu.sync_copy(buf, shared.at[pl.ds(s * AZT, AZT)])

    @pl.loop(0, K)
    def _ones(i):
        buf[i, :] = jnp.ones((16,), jnp.float32)

    pltpu.sync_copy(dst_hbm.at[s], dstv)

    @pl.loop(0, NCH2)
    def _remap(ci):
        @pl.loop(0, K, step=16)
        def _rl(j):
            v = dstv[ci, pl.ds(j, 16)] - base
            ok = (v >= 0) & (v < NH)
            dstv[ci, pl.ds(j, 16)] = jnp.where(ok, v, NH)

    plsc.subcore_barrier()

    @pl.loop(0, NCH2)
    def _scatter(ci):
        pltpu.sync_copy(buf.at[pl.ds(0, K)], shared.at[dstv.at[ci]], add=True)

    plsc.subcore_barrier()
    pltpu.sync_copy(shared.at[pl.ds(s * RTW, RTW)], out_hbm.at[c, pl.ds(s * RTW, RTW)])


def _agg_body(g_hbm, src_hbm, dst_hbm, out_hbm, srcv, dstv, rows0, rows1,
              shared, sem0, sem1):
    c = lax.axis_index("c")
    s = lax.axis_index("s")
    base = c * NH

    @pl.loop(0, K)
    def _zero(i):
        @pl.loop(0, D, step=16)
        def _zl(j):
            rows0[i, pl.ds(j, 16)] = jnp.zeros((16,), jnp.float32)

    @pl.loop(0, AZT // 36)
    def _zcopy(t):
        pltpu.sync_copy(rows0.at[pl.ds(0, 36)], shared.at[pl.ds(s * AZT + t * 36, 36)])

    pltpu.sync_copy(src_hbm.at[s], srcv)
    pltpu.sync_copy(dst_hbm.at[s], dstv)

    @pl.loop(0, NCH2)
    def _remap(ci):
        @pl.loop(0, K, step=16)
        def _rl(j):
            v = dstv[ci, pl.ds(j, 16)] - base
            ok = (v >= 0) & (v < NH)
            dstv[ci, pl.ds(j, 16)] = jnp.where(ok, v, NH)

    plsc.subcore_barrier()

    pltpu.async_copy(g_hbm.at[srcv.at[0]], rows0, sem0)

    @pl.loop(0, NCH2, step=2)
    def _pipe(ci):
        pltpu.async_copy(g_hbm.at[srcv.at[ci + 1]], rows1, sem1)
        pltpu.make_async_copy(g_hbm.at[srcv.at[ci]], rows0, sem0).wait()
        pltpu.sync_copy(rows0, shared.at[dstv.at[ci]], add=True)

        @pl.when(ci + 2 < NCH2)
        def _next():
            pltpu.async_copy(g_hbm.at[srcv.at[ci + 2]], rows0, sem0)

        pltpu.make_async_copy(g_hbm.at[srcv.at[ci + 1]], rows1, sem1).wait()
        pltpu.sync_copy(rows1, shared.at[dstv.at[ci + 1]], add=True)

    plsc.subcore_barrier()
    pltpu.sync_copy(shared.at[pl.ds(s * RTW, RTW)], out_hbm.at[c, pl.ds(s * RTW, RTW)])


def _sc_deg(dst2):
    mesh = plsc.VectorSubcoreMesh(core_axis_name="c", subcore_axis_name="s")
    f = pl.kernel(
        _deg_body,
        out_type=jax.ShapeDtypeStruct((NC, NH, 16), jnp.float32),
        mesh=mesh,
        scratch_types=[
            pltpu.VMEM((NCH2, K), jnp.int32),
            pltpu.VMEM((AZT, 16), jnp.float32),
            pltpu.VMEM_SHARED((ACC, 16), jnp.float32),
        ],
    )
    return f(dst2)


def _sc_agg(g, src3, dst3):
    mesh = plsc.VectorSubcoreMesh(core_axis_name="c", subcore_axis_name="s")
    f = pl.kernel(
        _agg_body,
        out_type=jax.ShapeDtypeStruct((NC, NH, D), jnp.float32),
        mesh=mesh,
        scratch_types=[
            pltpu.VMEM((NCH2, K), jnp.int32),
            pltpu.VMEM((NCH2, K), jnp.int32),
            pltpu.VMEM((K, D), jnp.float32),
            pltpu.VMEM((K, D), jnp.float32),
            pltpu.VMEM_SHARED((ACC, D), jnp.float32),
            pltpu.SemaphoreType.DMA,
            pltpu.SemaphoreType.DMA,
        ],
    )
    return f(g, src3, dst3)


def _mm_body(x_ref, w_ref, o_ref):
    o_ref[...] = jnp.dot(x_ref[...], w_ref[...], preferred_element_type=jnp.float32)


def _tc_matmul(x, w):
    return pl.pallas_call(
        _mm_body,
        grid=(GRID,),
        in_specs=[
            pl.BlockSpec((BR, D), lambda i: (i, 0)),
            pl.BlockSpec((D, D), lambda i: (0, 0)),
        ],
        out_specs=pl.BlockSpec((BR, D), lambda i: (i, 0)),
        out_shape=jax.ShapeDtypeStruct((NPAD, D), jnp.float32),
    )(x, w)


def _scale_body(deg_ref, h_ref, g_ref, dinv_ref):
    d = deg_ref[...] + 1.0
    dinv = lax.rsqrt(d)
    dinv_ref[...] = dinv
    g_ref[...] = h_ref[...] * dinv


def _tc_scale(degp, h):
    return pl.pallas_call(
        _scale_body,
        grid=(GRID,),
        in_specs=[
            pl.BlockSpec((BR, 1), lambda i: (i, 0)),
            pl.BlockSpec((BR, D), lambda i: (i, 0)),
        ],
        out_specs=[
            pl.BlockSpec((BR, D), lambda i: (i, 0)),
            pl.BlockSpec((BR, 1), lambda i: (i, 0)),
        ],
        out_shape=[
            jax.ShapeDtypeStruct((NPAD, D), jnp.float32),
            jax.ShapeDtypeStruct((NPAD, 1), jnp.float32),
        ],
    )(degp, h)


def _comb_body(a_ref, g_ref, dinv_ref, b_ref, w_ref, o_ref):
    dinv = dinv_ref[...]
    t = jax.nn.relu(dinv * (a_ref[...] + g_ref[...]) + b_ref[...])
    o_ref[...] = jnp.dot(t, w_ref[...], preferred_element_type=jnp.float32) * dinv


def _tc_combine_mm(agg, g, dinv2d, b, w):
    return pl.pallas_call(
        _comb_body,
        grid=(GRID,),
        in_specs=[
            pl.BlockSpec((BR, D), lambda i: (i, 0)),
            pl.BlockSpec((BR, D), lambda i: (i, 0)),
            pl.BlockSpec((BR, 1), lambda i: (i, 0)),
            pl.BlockSpec((1, D), lambda i: (0, 0)),
            pl.BlockSpec((D, D), lambda i: (0, 0)),
        ],
        out_specs=pl.BlockSpec((BR, D), lambda i: (i, 0)),
        out_shape=jax.ShapeDtypeStruct((NPAD, D), jnp.float32),
    )(agg, g, dinv2d, b, w)


def _final_body(a_ref, g_ref, dinv_ref, b_ref, batch_ref, h_ref, ge_ref):
    h = jax.nn.relu(dinv_ref[...] * (a_ref[...] + g_ref[...]) + b_ref[...])
    h_ref[...] = h
    ids = lax.broadcasted_iota(jnp.int32, (G, BR), 0)
    onehot = jnp.where(ids == batch_ref[...], 1.0, 0.0)
    contrib = jnp.dot(onehot, h, preferred_element_type=jnp.float32)
    i = pl.program_id(0)

    @pl.when(i == 0)
    def _init():
        ge_ref[...] = contrib

    @pl.when(i != 0)
    def _acc():
        ge_ref[...] += contrib


def _tc_final(agg, g, dinv2d, b, batch2d):
    return pl.pallas_call(
        _final_body,
        grid=(GRID,),
        in_specs=[
            pl.BlockSpec((BR, D), lambda i: (i, 0)),
            pl.BlockSpec((BR, D), lambda i: (i, 0)),
            pl.BlockSpec((BR, 1), lambda i: (i, 0)),
            pl.BlockSpec((1, D), lambda i: (0, 0)),
            pl.BlockSpec((1, BR), lambda i: (0, i)),
        ],
        out_specs=[
            pl.BlockSpec((BR, D), lambda i: (i, 0)),
            pl.BlockSpec((G, D), lambda i: (0, 0)),
        ],
        out_shape=[
            jax.ShapeDtypeStruct((NPAD, D), jnp.float32),
            jax.ShapeDtypeStruct((G, D), jnp.float32),
        ],
    )(agg, g, dinv2d, b, batch2d)


def kernel(x, edge_index, batch, W1, b1, W2, b2):
    src2 = edge_index[0].reshape(NS, NCH2, K)
    dst2 = edge_index[1].reshape(NS, NCH2, K)
    x_pad = jnp.pad(x, ((0, NPAD - N), (0, 0)))
    batch2d = jnp.pad(batch, (0, NPAD - N), constant_values=G).reshape(1, NPAD)
    b1r = b1.reshape(1, D)
    b2r = b2.reshape(1, D)

    degw = _sc_deg(dst2)
    h1 = _tc_matmul(x_pad, W1)

    degp = degw.reshape(NPAD, 16)[:, :1]
    g1, dinv2d = _tc_scale(degp, h1)

    agg1 = _sc_agg(g1, src2, dst2).reshape(NPAD, D)
    g2 = _tc_combine_mm(agg1, g1, dinv2d, b1r, W2)

    agg2 = _sc_agg(g2, src2, dst2).reshape(NPAD, D)
    h2full, ge = _tc_final(agg2, g2, dinv2d, b2r, batch2d)

    return h2full[:N], ge
